# Initial kernel scaffold; baseline (speedup 1.0000x reference)
import functools

import jax
import jax.numpy as jnp
from jax import lax
from jax.experimental import pallas as pl
from jax.experimental.pallas import tpu as pltpu
from jax.experimental.pallas import tpu_sc as plsc

N = 10000
E = 320000
B = 256
DH = 128
L = 3

NT = 16
CH = 128
NCH = 157
EPT = NCH * CH
EPAD = NT * EPT
NPAD = 10240
RPT = NPAD // NT
ZR = 32


def _seg_sum_pairs(x2, src2, dst2):
    mesh = plsc.VectorSubcoreMesh(core_axis_name="c", subcore_axis_name="s")

    @functools.partial(
        pl.kernel,
        out_type=jax.ShapeDtypeStruct((2, NPAD, DH), jnp.float32),
        mesh=mesh,
        scratch_types=[
            pltpu.VMEM((EPT,), jnp.int32),
            pltpu.VMEM((NCH, CH), jnp.int32),
            pltpu.VMEM((CH, DH), jnp.float32),
            pltpu.VMEM((ZR, DH), jnp.float32),
            pltpu.VMEM_SHARED((NPAD, DH), jnp.float32),
            pltpu.SemaphoreType.DMA,
        ],
    )
    def k(x_hbm, src_hbm, dst_hbm, out_hbm, src_v, dst_v, rows_v, zbuf, acc, sem):
        c = lax.axis_index("c")
        s = lax.axis_index("s")

        for r in range(ZR):
            for g in range(DH // 16):
                zbuf[r, pl.ds(g * 16, 16)] = jnp.zeros((16,), jnp.float32)
        row0 = s * RPT

        def zbody(i, _):
            pltpu.sync_copy(zbuf, acc.at[pl.ds(row0 + i * ZR, ZR)])
            return 0

        lax.fori_loop(0, RPT // ZR, zbody, 0)
        plsc.subcore_barrier()

        pltpu.sync_copy(src_hbm.at[c, s], src_v)
        pltpu.sync_copy(dst_hbm.at[c, s], dst_v)

        def body(j, _):
            idx = src_v.at[pl.ds(j * CH, CH)]
            pltpu.async_copy(x_hbm.at[c].at[idx], rows_v, sem).wait()
            pltpu.sync_copy(rows_v, acc.at[dst_v.at[j]], add=True)
            return 0

        lax.fori_loop(0, NCH, body, 0)
        plsc.subcore_barrier()

        pltpu.sync_copy(acc.at[pl.ds(row0, RPT)],
                        out_hbm.at[c, pl.ds(row0, RPT)])

    return k(x2, src2, dst2)


def _bn(h, g, b):
    m = jnp.mean(h, axis=0, keepdims=True)
    d = h - m
    v = jnp.mean(d * d, axis=0, keepdims=True)
    return g * d * lax.rsqrt(v + 1e-5) + b


def _lin0_body(x_ref, wt_ref, b_ref, o_ref):
    o_ref[0] = (
        jnp.dot(x_ref[0], wt_ref[0], preferred_element_type=jnp.float32)
        + b_ref[0]
    )


def _gin_dense(x, agg, eps, w1t, b1, g1, t1, w2t, b2, g2, t2):
    h = (1.0 + eps) * x + agg
    h = jnp.dot(h, w1t, preferred_element_type=jnp.float32) + b1
    h = jnp.maximum(_bn(h, g1, t1), 0.0)
    h = jnp.dot(h, w2t, preferred_element_type=jnp.float32) + b2
    return jnp.maximum(_bn(h, g2, t2), 0.0)


def _gin_body(x_ref, agg_ref, eps_ref, w1t_ref, b1_ref, g1_ref, t1_ref,
              w2t_ref, b2_ref, g2_ref, t2_ref, o_ref):
    o_ref[0] = _gin_dense(
        x_ref[0], agg_ref[0, :N, :], eps_ref[0], w1t_ref[0], b1_ref[0],
        g1_ref[0], t1_ref[0], w2t_ref[0], b2_ref[0], g2_ref[0], t2_ref[0])


def _final_body(x_ref, agg_ref, batch_ref, tm_ref, eps_ref, w1t_ref, b1_ref,
                g1_ref, t1_ref, w2t_ref, b2_ref, g2_ref, t2_ref,
                wa_ref, wb_ref, wc_ref, mb0_ref, mg0_ref, mt0_ref,
                mw1t_ref, mb1_ref, mg1_ref, mt1_ref, mw2t_ref, mb2_ref,
                o_ref):
    pools = []
    for t in range(2):
        h = _gin_dense(
            x_ref[t], agg_ref[t, :N, :], eps_ref[t], w1t_ref[t], b1_ref[t],
            g1_ref[t], t1_ref[t], w2t_ref[t], b2_ref[t], g2_ref[t], t2_ref[t])
        bt = batch_ref[t]
        oh = (bt == lax.broadcasted_iota(jnp.int32, (N, B), 1)).astype(
            jnp.float32)
        s = lax.dot_general(oh, h, (((0,), (0,)), ((), ())),
                            preferred_element_type=jnp.float32)
        cnt = lax.dot_general(oh, jnp.ones((N, 1), jnp.float32),
                              (((0,), (0,)), ((), ())),
                              preferred_element_type=jnp.float32)
        pools.append(s / jnp.maximum(cnt, 1.0))
    a = (
        jnp.dot(pools[0], wa_ref[...], preferred_element_type=jnp.float32)
        + jnp.dot(pools[1], wb_ref[...], preferred_element_type=jnp.float32)
        + jnp.dot(tm_ref[...], wc_ref[...], preferred_element_type=jnp.float32)
        + mb0_ref[...]
    )
    a = _bn(a, mg0_ref[...], mt0_ref[...])
    a = jnp.where(a >= 0.0, a, 0.01 * a)
    a = jnp.dot(a, mw1t_ref[...], preferred_element_type=jnp.float32) + mb1_ref[...]
    a = _bn(a, mg1_ref[...], mt1_ref[...])
    a = jnp.where(a >= 0.0, a, 0.01 * a)
    o_ref[...] = (
        jnp.dot(a, mw2t_ref[...], preferred_element_type=jnp.float32)
        + mb2_ref[...]
    )


def _tower_spec(shape):
    nd = len(shape)
    return pl.BlockSpec((1,) + shape, lambda t, _n=nd: (t,) + (0,) * _n)


def _pad_edges(edge_index):
    src = edge_index[0]
    dst = edge_index[1]
    npad = EPAD - E
    fill = jnp.arange(npad, dtype=jnp.int32)
    src = jnp.concatenate([src, fill % N])
    dst = jnp.concatenate([dst, N + fill % (NPAD - N)])
    return src.reshape(NT, EPT), dst.reshape(NT, NCH, CH)


def kernel(solute_x, solute_edge_index, solute_batch, solvent_x,
           solvent_edge_index, solvent_batch, tm,
           so_lin0_W, so_lin0_b, so_gin_W1, so_gin_b1, so_gin_W2, so_gin_b2,
           so_gin_g1, so_gin_bt1, so_gin_g2, so_gin_bt2, so_eps,
           sv_lin0_W, sv_lin0_b, sv_gin_W1, sv_gin_b1, sv_gin_W2, sv_gin_b2,
           sv_gin_g1, sv_gin_bt1, sv_gin_g2, sv_gin_bt2, sv_eps,
           mlp_W0, mlp_b0, mlp_g0, mlp_bt0, mlp_W1, mlp_b1, mlp_g1, mlp_bt1,
           mlp_W2, mlp_b2):
    f32 = jnp.float32

    x2 = jnp.stack([solute_x, solvent_x])
    so_src, so_dst = _pad_edges(solute_edge_index)
    sv_src, sv_dst = _pad_edges(solvent_edge_index)
    src2 = jnp.stack([so_src, sv_src])
    dst2 = jnp.stack([so_dst, sv_dst])
    batch2 = jnp.stack([solute_batch.reshape(N, 1),
                        solvent_batch.reshape(N, 1)])

    w0t = jnp.stack([so_lin0_W.T, sv_lin0_W.T])
    b0 = jnp.stack([so_lin0_b.reshape(1, DH), sv_lin0_b.reshape(1, DH)])

    def lw(i):
        return dict(
            eps=jnp.stack([so_eps[i].reshape(1, 1), sv_eps[i].reshape(1, 1)]),
            w1t=jnp.stack([so_gin_W1[i].T, sv_gin_W1[i].T]),
            b1=jnp.stack([so_gin_b1[i].reshape(1, DH), sv_gin_b1[i].reshape(1, DH)]),
            g1=jnp.stack([so_gin_g1[i].reshape(1, DH), sv_gin_g1[i].reshape(1, DH)]),
            t1=jnp.stack([so_gin_bt1[i].reshape(1, DH), sv_gin_bt1[i].reshape(1, DH)]),
            w2t=jnp.stack([so_gin_W2[i].T, sv_gin_W2[i].T]),
            b2=jnp.stack([so_gin_b2[i].reshape(1, DH), sv_gin_b2[i].reshape(1, DH)]),
            g2=jnp.stack([so_gin_g2[i].reshape(1, DH), sv_gin_g2[i].reshape(1, DH)]),
            t2=jnp.stack([so_gin_bt2[i].reshape(1, DH), sv_gin_bt2[i].reshape(1, DH)]),
        )

    mw0t = mlp_W0.T
    wa, wb, wc = mw0t[:DH], mw0t[DH:2 * DH], mw0t[2 * DH:]
    mw1t = mlp_W1.T
    mw2t = mlp_W2.T

    x2 = pl.pallas_call(
        _lin0_body,
        grid=(2,),
        in_specs=[_tower_spec((N, DH)), _tower_spec((DH, DH)),
                  _tower_spec((1, DH))],
        out_specs=_tower_spec((N, DH)),
        out_shape=jax.ShapeDtypeStruct((2, N, DH), f32),
    )(x2, w0t, b0)

    for i in range(L - 1):
        agg = _seg_sum_pairs(x2, src2, dst2)
        p = lw(i)
        x2 = pl.pallas_call(
            _gin_body,
            grid=(2,),
            in_specs=[_tower_spec((N, DH)), _tower_spec((NPAD, DH)),
                      _tower_spec((1, 1)), _tower_spec((DH, DH)),
                      _tower_spec((1, DH)), _tower_spec((1, DH)),
                      _tower_spec((1, DH)), _tower_spec((DH, DH)),
                      _tower_spec((1, DH)), _tower_spec((1, DH)),
                      _tower_spec((1, DH))],
            out_specs=_tower_spec((N, DH)),
            out_shape=jax.ShapeDtypeStruct((2, N, DH), f32),
        )(x2, agg, p["eps"], p["w1t"], p["b1"], p["g1"], p["t1"],
          p["w2t"], p["b2"], p["g2"], p["t2"])

    agg = _seg_sum_pairs(x2, src2, dst2)
    p = lw(L - 1)
    out = pl.pallas_call(
        _final_body,
        out_shape=jax.ShapeDtypeStruct((B, 1), f32),
    )(x2, agg, batch2, tm, p["eps"], p["w1t"], p["b1"], p["g1"], p["t1"],
      p["w2t"], p["b2"], p["g2"], p["t2"],
      wa, wb, wc, mlp_b0.reshape(1, 105), mlp_g0.reshape(1, 105),
      mlp_bt0.reshape(1, 105), mw1t, mlp_b1.reshape(1, 74),
      mlp_g1.reshape(1, 74), mlp_bt1.reshape(1, 74), mw2t,
      mlp_b2.reshape(1, 1))
    return out


# speedup vs baseline: 4.6718x; 4.6718x over previous
import functools

import jax
import jax.numpy as jnp
from jax import lax
from jax.experimental import pallas as pl
from jax.experimental.pallas import tpu as pltpu
from jax.experimental.pallas import tpu_sc as plsc

_PREC = lax.Precision.DEFAULT

N = 10000
E = 320000
B = 256
DH = 128
L = 3

NT = 16
CH = 128
KB = 8
NBLK = 20
EPT = NBLK * KB * CH
EPAD = NT * EPT
NPAD = 10240
RPT = NPAD // NT
ZR = 32


def _seg_sum_pairs(x2, src2, dst2):
    mesh = plsc.VectorSubcoreMesh(core_axis_name="c", subcore_axis_name="s")

    @functools.partial(
        pl.kernel,
        out_type=jax.ShapeDtypeStruct((2, NPAD, DH), jnp.float32),
        mesh=mesh,
        scratch_types=[
            pltpu.VMEM((KB, CH), jnp.int32),
            pltpu.VMEM((KB, CH), jnp.int32),
            pltpu.VMEM((CH, DH), jnp.float32),
            pltpu.VMEM((ZR, DH), jnp.float32),
            pltpu.VMEM_SHARED((NPAD, DH), jnp.float32),
            pltpu.SemaphoreType.DMA,
        ],
    )
    def k(x_hbm, src_hbm, dst_hbm, out_hbm, src_v, dst_v, rows_v, zbuf, acc, sem):
        c = lax.axis_index("c")
        s = lax.axis_index("s")

        for r in range(ZR):
            for g in range(DH // 16):
                zbuf[r, pl.ds(g * 16, 16)] = jnp.zeros((16,), jnp.float32)
        row0 = s * RPT

        def zbody(i, _):
            pltpu.sync_copy(zbuf, acc.at[pl.ds(row0 + i * ZR, ZR)])
            return 0

        lax.fori_loop(0, RPT // ZR, zbody, 0)
        plsc.subcore_barrier()

        def body(blk, _):
            pltpu.sync_copy(src_hbm.at[c, s, blk], src_v)
            pltpu.sync_copy(dst_hbm.at[c, s, blk], dst_v)
            for j in range(KB):
                pltpu.async_copy(x_hbm.at[c].at[src_v.at[j]], rows_v,
                                 sem).wait()
                pltpu.sync_copy(rows_v, acc.at[dst_v.at[j]], add=True)
            return 0

        lax.fori_loop(0, NBLK, body, 0)
        plsc.subcore_barrier()

        pltpu.sync_copy(acc.at[pl.ds(row0, RPT)],
                        out_hbm.at[c, pl.ds(row0, RPT)])

    return k(x2, src2, dst2)


def _bn(h, g, b):
    m = jnp.mean(h, axis=0, keepdims=True)
    d = h - m
    v = jnp.mean(d * d, axis=0, keepdims=True)
    return g * d / jnp.sqrt(v + 1e-5) + b


def _lin0_body(x_ref, wt_ref, b_ref, o_ref):
    o_ref[0] = (
        jnp.dot(x_ref[0], wt_ref[0], preferred_element_type=jnp.float32, precision=_PREC)
        + b_ref[0]
    )


def _gin_dense(x, agg, eps, w1t, b1, g1, t1, w2t, b2, g2, t2):
    h = (1.0 + eps) * x + agg
    h = jnp.dot(h, w1t, preferred_element_type=jnp.float32, precision=_PREC) + b1
    h = jnp.maximum(_bn(h, g1, t1), 0.0)
    h = jnp.dot(h, w2t, preferred_element_type=jnp.float32, precision=_PREC) + b2
    return jnp.maximum(_bn(h, g2, t2), 0.0)


def _gin_body(x_ref, agg_ref, eps_ref, w1t_ref, b1_ref, g1_ref, t1_ref,
              w2t_ref, b2_ref, g2_ref, t2_ref, o_ref):
    o_ref[0] = _gin_dense(
        x_ref[0], agg_ref[0, :N, :], eps_ref[0], w1t_ref[0], b1_ref[0],
        g1_ref[0], t1_ref[0], w2t_ref[0], b2_ref[0], g2_ref[0], t2_ref[0])


def _final_body(x_ref, agg_ref, batch_ref, tm_ref, eps_ref, w1t_ref, b1_ref,
                g1_ref, t1_ref, w2t_ref, b2_ref, g2_ref, t2_ref,
                wa_ref, wb_ref, wc_ref, mb0_ref, mg0_ref, mt0_ref,
                mw1t_ref, mb1_ref, mg1_ref, mt1_ref, mw2t_ref, mb2_ref,
                o_ref):
    pools = []
    for t in range(2):
        h = _gin_dense(
            x_ref[t], agg_ref[t, :N, :], eps_ref[t], w1t_ref[t], b1_ref[t],
            g1_ref[t], t1_ref[t], w2t_ref[t], b2_ref[t], g2_ref[t], t2_ref[t])
        bt = batch_ref[t]
        oh = (bt == lax.broadcasted_iota(jnp.int32, (N, B), 1)).astype(
            jnp.float32)
        s = lax.dot_general(oh, h, (((0,), (0,)), ((), ())),
                            preferred_element_type=jnp.float32,
                            precision=lax.Precision.HIGHEST)
        cnt = lax.dot_general(oh, jnp.ones((N, 1), jnp.float32),
                              (((0,), (0,)), ((), ())),
                              preferred_element_type=jnp.float32,
                              precision=lax.Precision.HIGHEST)
        pools.append(s / jnp.maximum(cnt, 1.0))
    a = (
        jnp.dot(pools[0], wa_ref[...], preferred_element_type=jnp.float32, precision=_PREC)
        + jnp.dot(pools[1], wb_ref[...], preferred_element_type=jnp.float32, precision=_PREC)
        + jnp.dot(tm_ref[...], wc_ref[...], preferred_element_type=jnp.float32, precision=_PREC)
        + mb0_ref[...]
    )
    a = _bn(a, mg0_ref[...], mt0_ref[...])
    a = jnp.where(a >= 0.0, a, 0.01 * a)
    a = jnp.dot(a, mw1t_ref[...], preferred_element_type=jnp.float32, precision=_PREC) + mb1_ref[...]
    a = _bn(a, mg1_ref[...], mt1_ref[...])
    a = jnp.where(a >= 0.0, a, 0.01 * a)
    o_ref[...] = (
        jnp.dot(a, mw2t_ref[...], preferred_element_type=jnp.float32, precision=_PREC)
        + mb2_ref[...]
    )


def _tower_spec(shape):
    nd = len(shape)
    return pl.BlockSpec((1,) + shape, lambda t, _n=nd: (t,) + (0,) * _n)


def _pad_edges(edge_index):
    src = edge_index[0]
    dst = edge_index[1]
    npad = EPAD - E
    fill = jnp.arange(npad, dtype=jnp.int32)
    src = jnp.concatenate([src, fill % N])
    dst = jnp.concatenate([dst, N + fill % (NPAD - N)])
    return (src.reshape(NT, NBLK, KB, CH), dst.reshape(NT, NBLK, KB, CH))


def kernel(solute_x, solute_edge_index, solute_batch, solvent_x,
           solvent_edge_index, solvent_batch, tm,
           so_lin0_W, so_lin0_b, so_gin_W1, so_gin_b1, so_gin_W2, so_gin_b2,
           so_gin_g1, so_gin_bt1, so_gin_g2, so_gin_bt2, so_eps,
           sv_lin0_W, sv_lin0_b, sv_gin_W1, sv_gin_b1, sv_gin_W2, sv_gin_b2,
           sv_gin_g1, sv_gin_bt1, sv_gin_g2, sv_gin_bt2, sv_eps,
           mlp_W0, mlp_b0, mlp_g0, mlp_bt0, mlp_W1, mlp_b1, mlp_g1, mlp_bt1,
           mlp_W2, mlp_b2):
    f32 = jnp.float32

    x2 = jnp.stack([solute_x, solvent_x])
    so_src, so_dst = _pad_edges(solute_edge_index)
    sv_src, sv_dst = _pad_edges(solvent_edge_index)
    src2 = jnp.stack([so_src, sv_src])
    dst2 = jnp.stack([so_dst, sv_dst])
    batch2 = jnp.stack([solute_batch.reshape(N, 1),
                        solvent_batch.reshape(N, 1)])

    w0t = jnp.stack([so_lin0_W.T, sv_lin0_W.T])
    b0 = jnp.stack([so_lin0_b.reshape(1, DH), sv_lin0_b.reshape(1, DH)])

    def lw(i):
        return dict(
            eps=jnp.stack([so_eps[i].reshape(1, 1), sv_eps[i].reshape(1, 1)]),
            w1t=jnp.stack([so_gin_W1[i].T, sv_gin_W1[i].T]),
            b1=jnp.stack([so_gin_b1[i].reshape(1, DH), sv_gin_b1[i].reshape(1, DH)]),
            g1=jnp.stack([so_gin_g1[i].reshape(1, DH), sv_gin_g1[i].reshape(1, DH)]),
            t1=jnp.stack([so_gin_bt1[i].reshape(1, DH), sv_gin_bt1[i].reshape(1, DH)]),
            w2t=jnp.stack([so_gin_W2[i].T, sv_gin_W2[i].T]),
            b2=jnp.stack([so_gin_b2[i].reshape(1, DH), sv_gin_b2[i].reshape(1, DH)]),
            g2=jnp.stack([so_gin_g2[i].reshape(1, DH), sv_gin_g2[i].reshape(1, DH)]),
            t2=jnp.stack([so_gin_bt2[i].reshape(1, DH), sv_gin_bt2[i].reshape(1, DH)]),
        )

    mw0t = mlp_W0.T
    wa, wb, wc = mw0t[:DH], mw0t[DH:2 * DH], mw0t[2 * DH:]
    mw1t = mlp_W1.T
    mw2t = mlp_W2.T

    x2 = pl.pallas_call(
        _lin0_body,
        grid=(2,),
        in_specs=[_tower_spec((N, DH)), _tower_spec((DH, DH)),
                  _tower_spec((1, DH))],
        out_specs=_tower_spec((N, DH)),
        out_shape=jax.ShapeDtypeStruct((2, N, DH), f32),
    )(x2, w0t, b0)

    for i in range(L - 1):
        agg = _seg_sum_pairs(x2, src2, dst2)
        p = lw(i)
        x2 = pl.pallas_call(
            _gin_body,
            grid=(2,),
            in_specs=[_tower_spec((N, DH)), _tower_spec((NPAD, DH)),
                      _tower_spec((1, 1)), _tower_spec((DH, DH)),
                      _tower_spec((1, DH)), _tower_spec((1, DH)),
                      _tower_spec((1, DH)), _tower_spec((DH, DH)),
                      _tower_spec((1, DH)), _tower_spec((1, DH)),
                      _tower_spec((1, DH))],
            out_specs=_tower_spec((N, DH)),
            out_shape=jax.ShapeDtypeStruct((2, N, DH), f32),
        )(x2, agg, p["eps"], p["w1t"], p["b1"], p["g1"], p["t1"],
          p["w2t"], p["b2"], p["g2"], p["t2"])

    agg = _seg_sum_pairs(x2, src2, dst2)
    p = lw(L - 1)
    out = pl.pallas_call(
        _final_body,
        out_shape=jax.ShapeDtypeStruct((B, 1), f32),
    )(x2, agg, batch2, tm, p["eps"], p["w1t"], p["b1"], p["g1"], p["t1"],
      p["w2t"], p["b2"], p["g2"], p["t2"],
      wa, wb, wc, mlp_b0.reshape(1, 105), mlp_g0.reshape(1, 105),
      mlp_bt0.reshape(1, 105), mw1t, mlp_b1.reshape(1, 74),
      mlp_g1.reshape(1, 74), mlp_bt1.reshape(1, 74), mw2t,
      mlp_b2.reshape(1, 1))
    return out
